# initial kernel scaffold (unmeasured)
import jax
import jax.numpy as jnp
from jax import lax
from jax.experimental import pallas as pl
from jax.experimental.pallas import tpu as pltpu

N_DEV = 32


def kernel(x, w_mat, scale_x, scale_w):
    m_per, k = x.shape
    _, n = w_mat.shape
    n_per = n // N_DEV

    def body(x_ref, w_ref, sx_ref, sw_ref, out_ref, acc_ref, send_sems, recv_sems):
        my = lax.axis_index("i")

        barrier = pltpu.get_barrier_semaphore()
        for p in range(N_DEV):
            pl.semaphore_signal(
                barrier, inc=1, device_id=(p,),
                device_id_type=pl.DeviceIdType.MESH,
            )
        pl.semaphore_wait(barrier, N_DEV)

        a = x_ref[...].astype(jnp.float8_e5m2)
        b = w_ref[...].astype(jnp.float8_e5m2)
        acc = lax.dot_general(
            a, b, (((1,), (0,)), ((), ())),
            preferred_element_type=jnp.float32,
        )
        s = sx_ref[0] * sw_ref[0]
        acc_ref[...] = jnp.maximum(acc * s, 0.0)

        out_ref[pl.ds(my * m_per, m_per), :] = acc_ref[:, pl.ds(my * n_per, n_per)]

        sends = []
        for t in range(1, N_DEV):
            d = lax.rem(my + t, N_DEV)
            rdma = pltpu.make_async_remote_copy(
                src_ref=acc_ref.at[:, pl.ds(d * n_per, n_per)],
                dst_ref=out_ref.at[pl.ds(my * m_per, m_per), :],
                send_sem=send_sems.at[t],
                recv_sem=recv_sems.at[t],
                device_id=(d,),
                device_id_type=pl.DeviceIdType.MESH,
            )
            rdma.start()
            sends.append(rdma)

        for t in range(1, N_DEV):
            src = lax.rem(my - t + N_DEV, N_DEV)
            recv = pltpu.make_async_remote_copy(
                src_ref=acc_ref.at[:, pl.ds(src * n_per, n_per)],
                dst_ref=out_ref.at[pl.ds(src * m_per, m_per), :],
                send_sem=send_sems.at[t],
                recv_sem=recv_sems.at[t],
                device_id=(src,),
                device_id_type=pl.DeviceIdType.MESH,
            )
            recv.wait_recv()

        for rdma in sends:
            rdma.wait_send()

    out_shape = jax.ShapeDtypeStruct((N_DEV * m_per, n_per), jnp.float32)
    return pl.pallas_call(
        body,
        out_shape=out_shape,
        in_specs=[
            pl.BlockSpec(memory_space=pltpu.VMEM),
            pl.BlockSpec(memory_space=pltpu.VMEM),
            pl.BlockSpec(memory_space=pltpu.SMEM),
            pl.BlockSpec(memory_space=pltpu.SMEM),
        ],
        out_specs=pl.BlockSpec(memory_space=pltpu.VMEM),
        scratch_shapes=[
            pltpu.VMEM((m_per, n), jnp.float32),
            pltpu.SemaphoreType.DMA((N_DEV,)),
            pltpu.SemaphoreType.DMA((N_DEV,)),
        ],
        compiler_params=pltpu.CompilerParams(
            collective_id=0,
            vmem_limit_bytes=100 * 1024 * 1024,
        ),
    )(x, w_mat, scale_x, scale_w)


# baseline (device time: 50036 ns/iter reference)
import jax
import jax.numpy as jnp
from jax import lax
from jax.experimental import pallas as pl
from jax.experimental.pallas import tpu as pltpu

N_DEV = 32


def kernel(x, w_mat, scale_x, scale_w):
    m_per, k = x.shape
    _, n = w_mat.shape
    n_per = n // N_DEV

    def body(x_ref, w_ref, sx_ref, sw_ref, out_ref, blk_ref, send_sems, recv_sems,
             local_sem):
        my = lax.axis_index("i")

        barrier = pltpu.get_barrier_semaphore()
        for p in range(N_DEV):
            pl.semaphore_signal(
                barrier, inc=1, device_id=(p,),
                device_id_type=pl.DeviceIdType.MESH,
            )
        pl.semaphore_wait(barrier, N_DEV)

        a = x_ref[...].astype(jnp.float8_e5m2)
        b = w_ref[...].astype(jnp.float8_e5m2)
        acc = lax.dot_general(
            a, b, (((1,), (0,)), ((), ())),
            preferred_element_type=jnp.float32,
        )
        s = sx_ref[0] * sw_ref[0]
        y = jnp.maximum(acc * s, 0.0)

        for d in range(N_DEV):
            blk_ref[d] = lax.slice(y, (0, d * n_per), (m_per, (d + 1) * n_per))

        own = pltpu.make_async_copy(
            blk_ref.at[my],
            out_ref.at[pl.ds(my * m_per, m_per), :],
            local_sem,
        )
        own.start()

        sends = []
        for t in range(1, N_DEV):
            d = lax.rem(my + t, N_DEV)
            rdma = pltpu.make_async_remote_copy(
                src_ref=blk_ref.at[d],
                dst_ref=out_ref.at[pl.ds(my * m_per, m_per), :],
                send_sem=send_sems.at[t],
                recv_sem=recv_sems.at[t],
                device_id=(d,),
                device_id_type=pl.DeviceIdType.MESH,
            )
            rdma.start()
            sends.append(rdma)

        for t in range(1, N_DEV):
            src = lax.rem(my - t + N_DEV, N_DEV)
            recv = pltpu.make_async_remote_copy(
                src_ref=blk_ref.at[0],
                dst_ref=out_ref.at[pl.ds(src * m_per, m_per), :],
                send_sem=send_sems.at[t],
                recv_sem=recv_sems.at[t],
                device_id=(src,),
                device_id_type=pl.DeviceIdType.MESH,
            )
            recv.wait_recv()

        own.wait()
        for rdma in sends:
            rdma.wait_send()

    out_shape = jax.ShapeDtypeStruct((N_DEV * m_per, n_per), jnp.float32)
    return pl.pallas_call(
        body,
        out_shape=out_shape,
        in_specs=[
            pl.BlockSpec(memory_space=pltpu.VMEM),
            pl.BlockSpec(memory_space=pltpu.VMEM),
            pl.BlockSpec(memory_space=pltpu.SMEM),
            pl.BlockSpec(memory_space=pltpu.SMEM),
        ],
        out_specs=pl.BlockSpec(memory_space=pltpu.VMEM),
        scratch_shapes=[
            pltpu.VMEM((N_DEV, m_per, n_per), jnp.float32),
            pltpu.SemaphoreType.DMA((N_DEV,)),
            pltpu.SemaphoreType.DMA((N_DEV,)),
            pltpu.SemaphoreType.DMA,
        ],
        compiler_params=pltpu.CompilerParams(
            collective_id=0,
            vmem_limit_bytes=100 * 1024 * 1024,
        ),
    )(x, w_mat, scale_x, scale_w)


# device time: 27806 ns/iter; 1.7995x vs baseline; 1.7995x over previous
import jax
import jax.numpy as jnp
from jax import lax
from jax.experimental import pallas as pl
from jax.experimental.pallas import tpu as pltpu

N_DEV = 32
N_PANELS = 8
DPP = N_DEV // N_PANELS


def kernel(x, w_mat, scale_x, scale_w):
    m_per, k = x.shape
    _, n = w_mat.shape
    n_per = n // N_DEV
    n_panel = n // N_PANELS

    def body(x_ref, w_ref, sx_ref, sw_ref, out_ref, x8_ref, wbuf_ref,
             blk_ref, send_sems, recv_sems, local_sem, wdma_sems,
             ready_sems):
        my = lax.axis_index("i")
        g = pl.program_id(0)
        panel = lax.rem(g + my // DPP, N_PANELS)

        bar = pltpu.get_barrier_semaphore()

        @pl.when(g == 0)
        def _():
            pl.semaphore_signal(
                bar, inc=1, device_id=(my,),
                device_id_type=pl.DeviceIdType.MESH,
            )
            pl.semaphore_wait(bar, 1)

        def wdma(p, slot):
            return pltpu.make_async_copy(
                w_ref.at[:, pl.ds(p * n_panel, n_panel)],
                wbuf_ref.at[slot],
                wdma_sems.at[slot],
            )

        @pl.when(g == 0)
        def _():
            for p in range(N_DEV):
                pl.semaphore_signal(
                    ready_sems.at[my], inc=1, device_id=(p,),
                    device_id_type=pl.DeviceIdType.MESH,
                )
            wdma(panel, 0).start()
            wdma(lax.rem(panel + 1, N_PANELS), 1).start()
            x8_ref[...] = x_ref[...].astype(jnp.float8_e5m2)

        @pl.when(jnp.logical_and(g >= 1, g <= N_PANELS - 2))
        def _():
            wdma(lax.rem(panel + 1, N_PANELS), lax.rem(g + 1, 2)).start()

        wdma(panel, lax.rem(g, 2)).wait()

        acc = lax.dot_general(
            x8_ref[...],
            wbuf_ref[lax.rem(g, 2)].astype(jnp.float8_e5m2),
            (((1,), (0,)), ((), ())),
            preferred_element_type=jnp.float32,
        )
        y = jnp.maximum(acc * (sx_ref[0] * sw_ref[0]), 0.0).astype(jnp.bfloat16)

        for j in range(DPP):
            blk_ref[panel * DPP + j] = lax.slice(
                y, (0, j * n_per), (m_per, (j + 1) * n_per))

        for j in range(DPP):
            d = panel * DPP + j

            @pl.when(d == my)
            def _(d=d):
                pltpu.make_async_copy(
                    blk_ref.at[d],
                    out_ref.at[pl.ds(d * m_per, m_per), :],
                    local_sem,
                ).start()

            @pl.when(d != my)
            def _(d=d):
                pl.semaphore_wait(ready_sems.at[d], 1)
                pltpu.make_async_remote_copy(
                    src_ref=blk_ref.at[d],
                    dst_ref=out_ref.at[pl.ds(my * m_per, m_per), :],
                    send_sem=send_sems.at[d],
                    recv_sem=recv_sems.at[my],
                    device_id=(d,),
                    device_id_type=pl.DeviceIdType.MESH,
                ).start()

        @pl.when(g == N_PANELS - 1)
        def _():
            pl.semaphore_wait(ready_sems.at[my], 1)
            pltpu.make_async_copy(
                blk_ref.at[my],
                out_ref.at[pl.ds(my * m_per, m_per), :],
                local_sem,
            ).wait()
            for t in range(1, N_DEV):
                s = lax.rem(my + t, N_DEV)
                pltpu.make_async_remote_copy(
                    src_ref=blk_ref.at[0],
                    dst_ref=out_ref.at[pl.ds(s * m_per, m_per), :],
                    send_sem=send_sems.at[s],
                    recv_sem=recv_sems.at[s],
                    device_id=(s,),
                    device_id_type=pl.DeviceIdType.MESH,
                ).wait_recv()
                pltpu.make_async_remote_copy(
                    src_ref=blk_ref.at[s],
                    dst_ref=out_ref.at[pl.ds(my * m_per, m_per), :],
                    send_sem=send_sems.at[s],
                    recv_sem=recv_sems.at[my],
                    device_id=(s,),
                    device_id_type=pl.DeviceIdType.MESH,
                ).wait_send()

    out = pl.pallas_call(
        body,
        grid=(N_PANELS,),
        out_shape=jax.ShapeDtypeStruct((N_DEV * m_per, n_per), jnp.bfloat16),
        in_specs=[
            pl.BlockSpec((m_per, k), lambda g: (0, 0)),
            pl.BlockSpec(memory_space=pl.ANY),
            pl.BlockSpec(memory_space=pltpu.SMEM),
            pl.BlockSpec(memory_space=pltpu.SMEM),
        ],
        out_specs=pl.BlockSpec(memory_space=pl.ANY),
        scratch_shapes=[
            pltpu.VMEM((m_per, k), jnp.float8_e5m2),
            pltpu.VMEM((2, k, n_panel), jnp.float32),
            pltpu.VMEM((N_DEV, m_per, n_per), jnp.bfloat16),
            pltpu.SemaphoreType.DMA((N_DEV,)),
            pltpu.SemaphoreType.DMA((N_DEV,)),
            pltpu.SemaphoreType.DMA,
            pltpu.SemaphoreType.DMA((2,)),
            pltpu.SemaphoreType.REGULAR((N_DEV,)),
        ],
        compiler_params=pltpu.CompilerParams(
            collective_id=0,
            vmem_limit_bytes=100 * 1024 * 1024,
            dimension_semantics=("arbitrary",),
        ),
    )(x, w_mat, scale_x, scale_w)
    return out.astype(jnp.float32)
